# Initial kernel scaffold; baseline (speedup 1.0000x reference)
import jax
import jax.numpy as jnp
from jax.experimental import pallas as pl


def kernel(X, hyperedges, W, b):
    raise NotImplementedError("write your pallas kernel here")


# speedup vs baseline: 87.3216x; 87.3216x over previous
import jax
import jax.numpy as jnp
from jax import lax
from jax.experimental import pallas as pl
from jax.experimental.pallas import tpu as pltpu
from jax.experimental.pallas import tpu_sc as plsc

V = 50000
F = 128
NUM_E = 2048
Q = 64
DUMMY = 50175
NPAD = 50176
CHUNK = 12544
NSUB = 16
NWORK = 32
TPT = (NUM_E * Q) // NSUB
IBLK = 2048
EPB = NUM_E // NWORK
EPAD = 2056
EID_PAD = 2048
BATCH = 64
STRIPE = CHUNK // NSUB
CBUF = TPT + 16


def _dedup_body(he_ref, out_ref):
    he = he_ref[...]
    q_iota = lax.broadcasted_iota(jnp.int32, (1, Q), 1)
    cols = [jnp.zeros((he.shape[0], 1), jnp.int32)]
    for q in range(1, Q):
        eq = (he == he[:, q:q + 1]) & (q_iota < q)
        cols.append(jnp.max(eq.astype(jnp.int32), axis=1, keepdims=True))
    dup = jnp.concatenate(cols, axis=1)
    out_ref[...] = jnp.where(dup == 1, DUMMY, he)


def _gather_mean_body(he_hbm, x_hbm, idxm_hbm, agg_hbm, deg_hbm,
                     idx_v, rows_v, blk_v, hist_v):
    wid = lax.axis_index("s") * 2 + lax.axis_index("c")
    e0 = wid * EPB
    pltpu.sync_copy(he_hbm.at[pl.ds(e0 * Q, EPB * Q)], idx_v)

    @pl.loop(0, EPB)
    def _(e):
        pltpu.sync_copy(x_hbm.at[idx_v.at[pl.ds(e * Q, Q)]], rows_v)

        def rbody(r, acc):
            return tuple(acc[j] + rows_v[r, pl.ds(j * 16, 16)]
                         for j in range(8))

        acc = lax.fori_loop(
            0, Q, rbody,
            tuple(jnp.zeros((16,), jnp.float32) for _ in range(8)))
        for j in range(8):
            blk_v[e, pl.ds(j * 16, 16)] = acc[j] * (1.0 / Q)

    pltpu.sync_copy(blk_v, agg_hbm.at[pl.ds(e0, EPB)])

    @pl.loop(0, NPAD // 16)
    def _(v):
        hist_v[pl.ds(v * 16, 16)] = jnp.zeros((16,), jnp.float32)

    pltpu.sync_copy(idxm_hbm.at[pl.ds(e0 * Q, EPB * Q)], idx_v)
    ones16 = jnp.ones((16,), jnp.float32)

    @pl.loop(0, (EPB * Q) // 16)
    def _(v):
        iv = idx_v[pl.ds(v * 16, 16)]
        plsc.addupdate_scatter(hist_v, [iv], ones16)

    pltpu.sync_copy(hist_v, deg_hbm.at[wid])


def _edge_matmul_body(agg_ref, w_ref, out_ref):
    aw = lax.dot_general(agg_ref[...], w_ref[...],
                         dimension_numbers=(((1,), (1,)), ((), ())),
                         preferred_element_type=jnp.float32)
    pad = jnp.zeros((EPAD - NUM_E, F), jnp.float32)
    out_ref[...] = jnp.concatenate([aw, pad], axis=0)


def _scatter_body(idx_hbm, aggw_hbm, zero_hbm, y0_hbm,
                  idx_v, pk_v, loc_st, eid_st, rows_v, acc_sh):
    cid = lax.axis_index("c")
    t = lax.axis_index("s")
    lane = lax.iota(jnp.int32, 16)

    for ci in range(2):
        base = (2 * cid + ci) * CHUNK
        pltpu.sync_copy(zero_hbm, acc_sh.at[pl.ds(t * STRIPE, STRIPE)])

        @pl.loop(0, CBUF // 16)
        def _(v):
            pk_v[pl.ds(v * 16, 16)] = jnp.full((16,), EID_PAD << 14,
                                               jnp.int32)

        plsc.subcore_barrier()

        def blk_scan(bi, off):
            pltpu.sync_copy(idx_hbm.at[pl.ds(t * TPT + bi * IBLK, IBLK)],
                            idx_v)

            def sbody(v, off):
                iv = idx_v[pl.ds(v * 16, 16)]
                loc = iv - base
                m = (loc >= 0) & (loc < CHUNK)
                g = t * TPT + bi * IBLK + v * 16 + lane
                eid = g // Q
                packed = loc | (eid << 14)
                plsc.store_compressed(pk_v.at[pl.ds(off, 16)], packed,
                                      mask=m)
                return off + jnp.max(plsc.all_reduce_population_count(m))

            return lax.fori_loop(0, IBLK // 16, sbody, off)

        n = lax.fori_loop(0, TPT // IBLK, blk_scan, jnp.int32(0))
        nb = (n + (BATCH - 1)) // BATCH

        def bbody(i, carry):
            for j in range(BATCH // 16):
                p = pk_v[pl.ds(i * BATCH + j * 16, 16)]
                loc_st[0, pl.ds(j * 16, 16)] = p & ((1 << 14) - 1)
                eid_st[0, pl.ds(j * 16, 16)] = lax.shift_right_logical(
                    p, 14)
            pltpu.sync_copy(aggw_hbm.at[eid_st.at[0]], rows_v)
            pltpu.sync_copy(rows_v, acc_sh.at[loc_st.at[0]], add=True)
            return carry

        lax.fori_loop(0, nb, bbody, jnp.int32(0))
        plsc.subcore_barrier()
        pltpu.sync_copy(acc_sh.at[pl.ds(t * STRIPE, STRIPE)],
                        y0_hbm.at[pl.ds(base + t * STRIPE, STRIPE)])
        plsc.subcore_barrier()


def _norm_body(y_ref, d_ref, b_ref, out_ref):
    y = y_ref[...]
    deg = jnp.sum(d_ref[...], axis=0, keepdims=True)
    deg = jnp.maximum(deg, 1.0).T
    out_ref[...] = (y / deg + b_ref[...])[None]


def kernel(X, hyperedges, W, b):
    x2 = X.reshape(V, F)
    he_flat = hyperedges.reshape(-1)

    idxm = pl.pallas_call(
        _dedup_body,
        grid=(8,),
        in_specs=[pl.BlockSpec((NUM_E // 8, Q), lambda i: (i, 0))],
        out_specs=pl.BlockSpec((NUM_E // 8, Q), lambda i: (i, 0)),
        out_shape=jax.ShapeDtypeStruct((NUM_E, Q), jnp.int32),
    )(hyperedges)
    idxm_flat = idxm.reshape(-1)

    agg, deg_part = pl.kernel(
        _gather_mean_body,
        out_type=[jax.ShapeDtypeStruct((NUM_E, F), jnp.float32),
                  jax.ShapeDtypeStruct((NWORK, NPAD), jnp.float32)],
        mesh=plsc.VectorSubcoreMesh(core_axis_name="c", subcore_axis_name="s"),
        compiler_params=pltpu.CompilerParams(needs_layout_passes=False),
        scratch_types=[pltpu.VMEM((EPB * Q,), jnp.int32),
                       pltpu.VMEM((Q, F), jnp.float32),
                       pltpu.VMEM((EPB, F), jnp.float32),
                       pltpu.VMEM((NPAD,), jnp.float32)],
    )(he_flat, x2, idxm_flat)

    aggw = pl.pallas_call(
        _edge_matmul_body,
        out_shape=jax.ShapeDtypeStruct((EPAD, F), jnp.float32),
    )(agg, W)

    y0 = pl.kernel(
        _scatter_body,
        out_type=jax.ShapeDtypeStruct((NPAD, F), jnp.float32),
        mesh=plsc.VectorSubcoreMesh(core_axis_name="c", subcore_axis_name="s"),
        compiler_params=pltpu.CompilerParams(needs_layout_passes=False),
        scratch_types=[pltpu.VMEM((IBLK,), jnp.int32),
                       pltpu.VMEM((CBUF,), jnp.int32),
                       pltpu.VMEM((1, BATCH), jnp.int32),
                       pltpu.VMEM((1, BATCH), jnp.int32),
                       pltpu.VMEM((BATCH, F), jnp.float32),
                       pltpu.VMEM_SHARED((CHUNK, F), jnp.float32)],
    )(idxm_flat, aggw, jnp.zeros((STRIPE, F), jnp.float32))

    out = pl.pallas_call(
        _norm_body,
        grid=(NPAD // 256,),
        in_specs=[pl.BlockSpec((256, F), lambda i: (i, 0)),
                  pl.BlockSpec((NWORK, 256), lambda i: (0, i)),
                  pl.BlockSpec((1, F), lambda i: (0, 0))],
        out_specs=pl.BlockSpec((1, 256, F), lambda i: (0, i, 0)),
        out_shape=jax.ShapeDtypeStruct((1, V, F), jnp.float32),
    )(y0, deg_part, b.reshape(1, F))
    return out
